# Initial kernel scaffold; baseline (speedup 1.0000x reference)
import jax
import jax.numpy as jnp
from jax.experimental import pallas as pl


def kernel(x, edge_index, edge_weight, attention, Wz, bz, Wr, br, Wh, bh, Lz_w, Lz_b, Lr_w, Lr_b, Lh_w, Lh_b, W_lin, b_lin):
    raise NotImplementedError("write your pallas kernel here")


# speedup vs baseline: 16.2274x; 16.2274x over previous
import functools

import jax
import jax.numpy as jnp
from jax import lax
from jax.experimental import pallas as pl
from jax.experimental.pallas import tpu as pltpu
from jax.experimental.pallas import tpu_sc as plsc

N = 10000
NP = 10240
F = 128
P = 12
FOUT = 256
E = 160000
EP = 163840
NS = 16
NC = 2
CH = EP // NS // 128
G = 128
NPT = NP // NS
PT = P // NC


def _sc_propagate(x_flat, srcr, dstr, wr, *, interpret=False):
    mesh = plsc.VectorSubcoreMesh(core_axis_name="c", subcore_axis_name="s",
                                  num_cores=NC, num_subcores=NS)

    @functools.partial(
        pl.kernel,
        out_type=jax.ShapeDtypeStruct((P * NP, F), jnp.float32),
        mesh=mesh,
        interpret=interpret,
        compiler_params=pltpu.CompilerParams(needs_layout_passes=False),
        scratch_types=[
            pltpu.VMEM((CH, G), jnp.float32),
            pltpu.VMEM((NP,), jnp.float32),
            pltpu.VMEM((G,), jnp.int32),
            pltpu.VMEM((G,), jnp.int32),
            pltpu.VMEM((G,), jnp.int32),
            pltpu.VMEM((G, F), jnp.float32),
            pltpu.VMEM((NPT,), jnp.float32),
            pltpu.VMEM_SHARED((NP,), jnp.float32),
            pltpu.VMEM_SHARED((NP, F), jnp.float32),
        ],
    )
    def k(x_hbm, src_hbm, dst_hbm, w_hbm, y_hbm,
          nrm_v, dinv_v, src_c, dst_c, idx_v, rows_v, zbuf, deg_sh, acc_sh):
        c = lax.axis_index("c")
        s = lax.axis_index("s")

        pltpu.sync_copy(w_hbm.at[s], nrm_v)

        for i in range(NPT // 16):
            zbuf[pl.ds(i * 16, 16)] = jnp.zeros((16,), jnp.float32)
        pltpu.sync_copy(zbuf, deg_sh.at[pl.ds(s * NPT, NPT)])
        plsc.subcore_barrier()

        def deg_body(j, _):
            pltpu.sync_copy(dst_hbm.at[s, j], dst_c)
            pltpu.sync_copy(nrm_v.at[j], deg_sh.at[dst_c], add=True)
            return 0
        lax.fori_loop(0, CH, deg_body, 0)
        plsc.subcore_barrier()

        pltpu.sync_copy(deg_sh, dinv_v)

        def dinv_body(i, _):
            d = dinv_v[pl.ds(i * 16, 16)] + 1.0
            bi = lax.bitcast_convert_type(d, jnp.int32)
            y0 = lax.bitcast_convert_type(
                0x5F3759DF - lax.shift_right_arithmetic(bi, 1), jnp.float32)
            hd = 0.5 * d
            y0 = y0 * (1.5 - hd * y0 * y0)
            y0 = y0 * (1.5 - hd * y0 * y0)
            y0 = y0 * (1.5 - hd * y0 * y0)
            dinv_v[pl.ds(i * 16, 16)] = y0
            return 0
        lax.fori_loop(0, NP // 16, dinv_body, 0)

        def nrm_body(j, _):
            pltpu.sync_copy(src_hbm.at[s, j], src_c)
            pltpu.sync_copy(dst_hbm.at[s, j], dst_c)
            for kk in range(G // 16):
                sl = pl.ds(kk * 16, 16)
                a = plsc.load_gather(dinv_v, [src_c[sl]])
                b = plsc.load_gather(dinv_v, [dst_c[sl]])
                nrm_v[j, sl] = a * nrm_v[j, sl] * b
            return 0
        lax.fori_loop(0, CH, nrm_body, 0)

        def period_body(tt, _):
            t = c * PT + tt
            tbase = t * NP
            base0 = s * NPT
            plsc.subcore_barrier()

            def init_chunk(c2, _):
                nb = base0 + c2 * G
                pltpu.sync_copy(x_hbm.at[pl.ds(tbase + nb, G)], rows_v)

                def init_row(r, _):
                    dv = plsc.load_gather(
                        dinv_v, [jnp.full((16,), nb + r, jnp.int32)])
                    cf = dv * dv
                    for kk in range(F // 16):
                        sl = pl.ds(kk * 16, 16)
                        rows_v[r, sl] = rows_v[r, sl] * cf
                    return 0
                lax.fori_loop(0, G, init_row, 0)
                pltpu.sync_copy(rows_v, acc_sh.at[pl.ds(nb, G)])
                return 0
            lax.fori_loop(0, NPT // G, init_chunk, 0)
            plsc.subcore_barrier()

            def edge_chunk(j, _):
                pltpu.sync_copy(src_hbm.at[s, j], src_c)
                pltpu.sync_copy(dst_hbm.at[s, j], dst_c)
                for kk in range(G // 16):
                    sl = pl.ds(kk * 16, 16)
                    idx_v[sl] = src_c[sl] + tbase
                pltpu.sync_copy(x_hbm.at[idx_v], rows_v)

                def scale_row(e2, _):
                    cf = plsc.load_gather(
                        nrm_v, [jnp.full((16,), j, jnp.int32),
                                jnp.full((16,), e2, jnp.int32)])
                    for kk in range(F // 16):
                        sl = pl.ds(kk * 16, 16)
                        rows_v[e2, sl] = rows_v[e2, sl] * cf
                    return 0
                lax.fori_loop(0, G, scale_row, 0)
                pltpu.sync_copy(rows_v, acc_sh.at[dst_c], add=True)
                return 0
            lax.fori_loop(0, CH, edge_chunk, 0)
            plsc.subcore_barrier()

            pltpu.sync_copy(acc_sh.at[pl.ds(base0, NPT)],
                            y_hbm.at[pl.ds(tbase + base0, NPT)])
            return 0
        lax.fori_loop(0, PT, period_body, 0)

    return k(x_flat, srcr, dstr, wr)


def _tc_gates(y, attention, Wz, Wh, Lzt, Lht, bz, bh, Lzb, Lhb, Wlin, blin,
              *, interpret=False):
    NB = 1024
    grid = (NP // NB, P)

    def body(att_ref, wz_ref, wh_ref, lzt_ref, lht_ref, bz_ref, bh_ref,
             lzb_ref, lhb_ref, wlin_ref, blin_ref, y_ref, out_ref,
             az_s, ah_s, cz_s, ch_s, h_s):
        nb = pl.program_id(0)
        t = pl.program_id(1)

        @pl.when((nb == 0) & (t == 0))
        def _():
            az_s[...] = jnp.dot(wz_ref[...], lzt_ref[...],
                                preferred_element_type=jnp.float32)
            ah_s[...] = jnp.dot(wh_ref[...], lht_ref[...],
                                preferred_element_type=jnp.float32)
            cz_s[...] = jnp.dot(bz_ref[...], lzt_ref[...],
                                preferred_element_type=jnp.float32) + lzb_ref[...]
            ch_s[...] = jnp.dot(bh_ref[...], lht_ref[...],
                                preferred_element_type=jnp.float32) + lhb_ref[...]

        att = att_ref[...]
        ex = jnp.exp(att - jnp.max(att))
        pr = ex / jnp.sum(ex)
        pt = jnp.sum(jnp.where(
            lax.broadcasted_iota(jnp.int32, (1, P), 1) == t, pr, 0.0))

        yb = y_ref[0]
        z = jax.nn.sigmoid(jnp.dot(yb, az_s[...],
                                   preferred_element_type=jnp.float32)
                           + cz_s[...])
        ht = jnp.tanh(jnp.dot(yb, ah_s[...],
                              preferred_element_type=jnp.float32) + ch_s[...])
        contrib = (pt * (1.0 - z)) * ht

        @pl.when(t == 0)
        def _():
            h_s[...] = contrib

        @pl.when(t > 0)
        def _():
            h_s[...] = h_s[...] + contrib

        @pl.when(t == P - 1)
        def _():
            out_ref[...] = jnp.dot(jnp.maximum(h_s[...], 0.0), wlin_ref[...],
                                   preferred_element_type=jnp.float32) + blin_ref[...]

    return pl.pallas_call(
        body,
        grid=grid,
        in_specs=[
            pl.BlockSpec((1, P), lambda nb, t: (0, 0)),
            pl.BlockSpec((F, FOUT), lambda nb, t: (0, 0)),
            pl.BlockSpec((F, FOUT), lambda nb, t: (0, 0)),
            pl.BlockSpec((FOUT, FOUT), lambda nb, t: (0, 0)),
            pl.BlockSpec((FOUT, FOUT), lambda nb, t: (0, 0)),
            pl.BlockSpec((1, FOUT), lambda nb, t: (0, 0)),
            pl.BlockSpec((1, FOUT), lambda nb, t: (0, 0)),
            pl.BlockSpec((1, FOUT), lambda nb, t: (0, 0)),
            pl.BlockSpec((1, FOUT), lambda nb, t: (0, 0)),
            pl.BlockSpec((FOUT, P), lambda nb, t: (0, 0)),
            pl.BlockSpec((1, P), lambda nb, t: (0, 0)),
            pl.BlockSpec((1, NB, F), lambda nb, t: (t, nb, 0)),
        ],
        out_specs=pl.BlockSpec((NB, P), lambda nb, t: (nb, 0)),
        out_shape=jax.ShapeDtypeStruct((NP, P), jnp.float32),
        scratch_shapes=[
            pltpu.VMEM((F, FOUT), jnp.float32),
            pltpu.VMEM((F, FOUT), jnp.float32),
            pltpu.VMEM((1, FOUT), jnp.float32),
            pltpu.VMEM((1, FOUT), jnp.float32),
            pltpu.VMEM((NB, FOUT), jnp.float32),
        ],
        interpret=interpret,
    )(attention.reshape(1, P), Wz, Wh, Lzt, Lht,
      bz.reshape(1, FOUT), bh.reshape(1, FOUT),
      Lzb.reshape(1, FOUT), Lhb.reshape(1, FOUT),
      Wlin, blin.reshape(1, P), y)


def kernel(x, edge_index, edge_weight, attention, Wz, bz, Wr, br, Wh, bh,
           Lz_w, Lz_b, Lr_w, Lr_b, Lh_w, Lh_b, W_lin, b_lin):
    xT = jnp.transpose(x, (2, 0, 1))
    xT = jnp.pad(xT, ((0, 0), (0, NP - N), (0, 0)))
    x_flat = xT.reshape(P * NP, F)

    src = jnp.pad(edge_index[0], (0, EP - E)).reshape(NS, CH, G)
    dst = jnp.pad(edge_index[1], (0, EP - E)).reshape(NS, CH, G)
    w = jnp.pad(edge_weight, (0, EP - E)).reshape(NS, CH, G)

    y = _sc_propagate(x_flat, src, dst, w).reshape(P, NP, F)
    out = _tc_gates(y, attention, Wz, Wh, Lz_w[:FOUT], Lh_w[:FOUT],
                    bz, bh, Lz_b, Lh_b, W_lin, b_lin)
    return out[:N]


# speedup vs baseline: 17.9506x; 1.1062x over previous
import functools

import jax
import jax.numpy as jnp
from jax import lax
from jax.experimental import pallas as pl
from jax.experimental.pallas import tpu as pltpu
from jax.experimental.pallas import tpu_sc as plsc

N = 10000
NP = 10240
F = 128
P = 12
FOUT = 256
E = 160000
EP = 163840
NS = 16
NC = 2
CH = EP // NS // 128
G = 128
NPT = NP // NS
PT = P // NC


def _sc_propagate(x_flat, srcr, dstr, wr, *, interpret=False):
    mesh = plsc.VectorSubcoreMesh(core_axis_name="c", subcore_axis_name="s",
                                  num_cores=NC, num_subcores=NS)

    @functools.partial(
        pl.kernel,
        out_type=jax.ShapeDtypeStruct((P * NP, F), jnp.float32),
        mesh=mesh,
        interpret=interpret,
        compiler_params=pltpu.CompilerParams(needs_layout_passes=False),
        scratch_types=[
            pltpu.VMEM((CH, G), jnp.float32),
            pltpu.VMEM((NP,), jnp.float32),
            pltpu.VMEM((G,), jnp.int32),
            pltpu.VMEM((G,), jnp.int32),
            pltpu.VMEM((G,), jnp.int32),
            pltpu.VMEM((G, F), jnp.float32),
            pltpu.VMEM((NPT,), jnp.float32),
            pltpu.VMEM_SHARED((NP,), jnp.float32),
            pltpu.VMEM_SHARED((NP, F), jnp.float32),
        ],
    )
    def k(x_hbm, src_hbm, dst_hbm, w_hbm, y_hbm,
          nrm_v, dinv_v, src_c, dst_c, idx_v, rows_v, zbuf, deg_sh, acc_sh):
        c = lax.axis_index("c")
        s = lax.axis_index("s")

        pltpu.sync_copy(w_hbm.at[s], nrm_v)

        for i in range(NPT // 16):
            zbuf[pl.ds(i * 16, 16)] = jnp.zeros((16,), jnp.float32)
        pltpu.sync_copy(zbuf, deg_sh.at[pl.ds(s * NPT, NPT)])
        plsc.subcore_barrier()

        def deg_body(j, _):
            pltpu.sync_copy(dst_hbm.at[s, j], dst_c)
            pltpu.sync_copy(nrm_v.at[j], deg_sh.at[dst_c], add=True)
            return 0
        lax.fori_loop(0, CH, deg_body, 0)
        plsc.subcore_barrier()

        pltpu.sync_copy(deg_sh, dinv_v)

        def dinv_body(i, _):
            d = dinv_v[pl.ds(i * 16, 16)] + 1.0
            bi = lax.bitcast_convert_type(d, jnp.int32)
            y0 = lax.bitcast_convert_type(
                0x5F3759DF - lax.shift_right_arithmetic(bi, 1), jnp.float32)
            hd = 0.5 * d
            y0 = y0 * (1.5 - hd * y0 * y0)
            y0 = y0 * (1.5 - hd * y0 * y0)
            y0 = y0 * (1.5 - hd * y0 * y0)
            dinv_v[pl.ds(i * 16, 16)] = y0
            return 0
        lax.fori_loop(0, NP // 16, dinv_body, 0)

        def nrm_body(j, _):
            pltpu.sync_copy(src_hbm.at[s, j], src_c)
            pltpu.sync_copy(dst_hbm.at[s, j], dst_c)
            for kk in range(G // 16):
                sl = pl.ds(kk * 16, 16)
                a = plsc.load_gather(dinv_v, [src_c[sl]])
                b = plsc.load_gather(dinv_v, [dst_c[sl]])
                nrm_v[j, sl] = a * nrm_v[j, sl] * b
            return 0
        lax.fori_loop(0, CH, nrm_body, 0)

        def period_body(tt, _):
            t = c * PT + tt
            tbase = t * NP
            base0 = s * NPT
            plsc.subcore_barrier()

            def init_chunk(c2, _):
                nb = base0 + c2 * G
                pltpu.sync_copy(x_hbm.at[pl.ds(tbase + nb, G)], rows_v)

                def init_rows16(q, _):
                    dv = dinv_v[pl.ds(nb + q * 16, 16)]
                    d2 = dv * dv
                    for r in range(16):
                        cf = jnp.broadcast_to(d2[r], (16,))
                        e2 = q * 16 + r
                        for kk in range(F // 16):
                            sl = pl.ds(kk * 16, 16)
                            rows_v[e2, sl] = rows_v[e2, sl] * cf
                    return 0
                lax.fori_loop(0, G // 16, init_rows16, 0)
                pltpu.sync_copy(rows_v, acc_sh.at[pl.ds(nb, G)])
                return 0
            lax.fori_loop(0, NPT // G, init_chunk, 0)
            plsc.subcore_barrier()

            def edge_chunk(j, _):
                pltpu.sync_copy(src_hbm.at[s, j], src_c)
                pltpu.sync_copy(dst_hbm.at[s, j], dst_c)
                for kk in range(G // 16):
                    sl = pl.ds(kk * 16, 16)
                    idx_v[sl] = src_c[sl] + tbase
                pltpu.sync_copy(x_hbm.at[idx_v], rows_v)

                def scale_rows16(q, _):
                    nv = nrm_v[j, pl.ds(q * 16, 16)]
                    for r in range(16):
                        cf = jnp.broadcast_to(nv[r], (16,))
                        e2 = q * 16 + r
                        for kk in range(F // 16):
                            sl = pl.ds(kk * 16, 16)
                            rows_v[e2, sl] = rows_v[e2, sl] * cf
                    return 0
                lax.fori_loop(0, G // 16, scale_rows16, 0)
                pltpu.sync_copy(rows_v, acc_sh.at[dst_c], add=True)
                return 0
            lax.fori_loop(0, CH, edge_chunk, 0)
            plsc.subcore_barrier()

            pltpu.sync_copy(acc_sh.at[pl.ds(base0, NPT)],
                            y_hbm.at[pl.ds(tbase + base0, NPT)])
            return 0
        lax.fori_loop(0, PT, period_body, 0)

    return k(x_flat, srcr, dstr, wr)


def _tc_gates(y, attention, Wz, Wh, Lzt, Lht, bz, bh, Lzb, Lhb, Wlin, blin,
              *, interpret=False):
    NB = 1024
    grid = (NP // NB, P)

    def body(att_ref, wz_ref, wh_ref, lzt_ref, lht_ref, bz_ref, bh_ref,
             lzb_ref, lhb_ref, wlin_ref, blin_ref, y_ref, out_ref,
             az_s, ah_s, cz_s, ch_s, h_s):
        nb = pl.program_id(0)
        t = pl.program_id(1)

        @pl.when((nb == 0) & (t == 0))
        def _():
            az_s[...] = jnp.dot(wz_ref[...], lzt_ref[...],
                                preferred_element_type=jnp.float32)
            ah_s[...] = jnp.dot(wh_ref[...], lht_ref[...],
                                preferred_element_type=jnp.float32)
            cz_s[...] = jnp.dot(bz_ref[...], lzt_ref[...],
                                preferred_element_type=jnp.float32) + lzb_ref[...]
            ch_s[...] = jnp.dot(bh_ref[...], lht_ref[...],
                                preferred_element_type=jnp.float32) + lhb_ref[...]

        att = att_ref[...]
        ex = jnp.exp(att - jnp.max(att))
        pr = ex / jnp.sum(ex)
        pt = jnp.sum(jnp.where(
            lax.broadcasted_iota(jnp.int32, (1, P), 1) == t, pr, 0.0))

        yb = y_ref[0]
        z = jax.nn.sigmoid(jnp.dot(yb, az_s[...],
                                   preferred_element_type=jnp.float32)
                           + cz_s[...])
        ht = jnp.tanh(jnp.dot(yb, ah_s[...],
                              preferred_element_type=jnp.float32) + ch_s[...])
        contrib = (pt * (1.0 - z)) * ht

        @pl.when(t == 0)
        def _():
            h_s[...] = contrib

        @pl.when(t > 0)
        def _():
            h_s[...] = h_s[...] + contrib

        @pl.when(t == P - 1)
        def _():
            out_ref[...] = jnp.dot(jnp.maximum(h_s[...], 0.0), wlin_ref[...],
                                   preferred_element_type=jnp.float32) + blin_ref[...]

    return pl.pallas_call(
        body,
        grid=grid,
        in_specs=[
            pl.BlockSpec((1, P), lambda nb, t: (0, 0)),
            pl.BlockSpec((F, FOUT), lambda nb, t: (0, 0)),
            pl.BlockSpec((F, FOUT), lambda nb, t: (0, 0)),
            pl.BlockSpec((FOUT, FOUT), lambda nb, t: (0, 0)),
            pl.BlockSpec((FOUT, FOUT), lambda nb, t: (0, 0)),
            pl.BlockSpec((1, FOUT), lambda nb, t: (0, 0)),
            pl.BlockSpec((1, FOUT), lambda nb, t: (0, 0)),
            pl.BlockSpec((1, FOUT), lambda nb, t: (0, 0)),
            pl.BlockSpec((1, FOUT), lambda nb, t: (0, 0)),
            pl.BlockSpec((FOUT, P), lambda nb, t: (0, 0)),
            pl.BlockSpec((1, P), lambda nb, t: (0, 0)),
            pl.BlockSpec((1, NB, F), lambda nb, t: (t, nb, 0)),
        ],
        out_specs=pl.BlockSpec((NB, P), lambda nb, t: (nb, 0)),
        out_shape=jax.ShapeDtypeStruct((NP, P), jnp.float32),
        scratch_shapes=[
            pltpu.VMEM((F, FOUT), jnp.float32),
            pltpu.VMEM((F, FOUT), jnp.float32),
            pltpu.VMEM((1, FOUT), jnp.float32),
            pltpu.VMEM((1, FOUT), jnp.float32),
            pltpu.VMEM((NB, FOUT), jnp.float32),
        ],
        interpret=interpret,
    )(attention.reshape(1, P), Wz, Wh, Lzt, Lht,
      bz.reshape(1, FOUT), bh.reshape(1, FOUT),
      Lzb.reshape(1, FOUT), Lhb.reshape(1, FOUT),
      Wlin, blin.reshape(1, P), y)


def kernel(x, edge_index, edge_weight, attention, Wz, bz, Wr, br, Wh, bh,
           Lz_w, Lz_b, Lr_w, Lr_b, Lh_w, Lh_b, W_lin, b_lin):
    xT = jnp.transpose(x, (2, 0, 1))
    xT = jnp.pad(xT, ((0, 0), (0, NP - N), (0, 0)))
    x_flat = xT.reshape(P * NP, F)

    src = jnp.pad(edge_index[0], (0, EP - E)).reshape(NS, CH, G)
    dst = jnp.pad(edge_index[1], (0, EP - E)).reshape(NS, CH, G)
    w = jnp.pad(edge_weight, (0, EP - E)).reshape(NS, CH, G)

    y = _sc_propagate(x_flat, src, dst, w).reshape(P, NP, F)
    out = _tc_gates(y, attention, Wz, Wh, Lz_w[:FOUT], Lh_w[:FOUT],
                    bz, bh, Lz_b, Lh_b, W_lin, b_lin)
    return out[:N]


# speedup vs baseline: 21.7898x; 1.2139x over previous
import functools

import jax
import jax.numpy as jnp
from jax import lax
from jax.experimental import pallas as pl
from jax.experimental.pallas import tpu as pltpu
from jax.experimental.pallas import tpu_sc as plsc

N = 10000
NP = 10240
F = 128
P = 12
FOUT = 256
E = 160000
EP = 163840
NS = 16
NC = 2
CH = EP // NS // 128
G = 128
NPT = NP // NS
PT = P // NC


def _sc_propagate(x_flat, srcr, dstr, wr, *, interpret=False):
    mesh = plsc.VectorSubcoreMesh(core_axis_name="c", subcore_axis_name="s",
                                  num_cores=NC, num_subcores=NS)

    @functools.partial(
        pl.kernel,
        out_type=jax.ShapeDtypeStruct((P * NP, F), jnp.float32),
        mesh=mesh,
        interpret=interpret,
        compiler_params=pltpu.CompilerParams(needs_layout_passes=False),
        scratch_types=[
            pltpu.VMEM((CH, G), jnp.float32),
            pltpu.VMEM((NPT,), jnp.float32),
            pltpu.VMEM((G, F), jnp.float32),
            pltpu.VMEM((G, F), jnp.float32),
            pltpu.VMEM((G,), jnp.int32),
            pltpu.VMEM((G,), jnp.int32),
            pltpu.VMEM((G,), jnp.int32),
            pltpu.VMEM((G,), jnp.int32),
            pltpu.VMEM((G,), jnp.int32),
            pltpu.VMEM((G,), jnp.int32),
            pltpu.VMEM((G,), jnp.float32),
            pltpu.VMEM((G,), jnp.float32),
            pltpu.VMEM((NPT,), jnp.float32),
            pltpu.VMEM_SHARED((NP,), jnp.float32),
            pltpu.VMEM_SHARED((NP,), jnp.float32),
            pltpu.VMEM_SHARED((NP, F), jnp.float32),
            pltpu.SemaphoreType.DMA,
            pltpu.SemaphoreType.DMA,
        ],
    )
    def k(x_hbm, src_hbm, dst_hbm, w_hbm, y_hbm,
          nrm_v, dinv2_v, rows0, rows1, src0, src1, dst0, dst1, idx0, idx1,
          a_v, b_v, zbuf, deg_sh, dinv_sh, acc_sh, semg0, semg1):
        c = lax.axis_index("c")
        s = lax.axis_index("s")
        rows = (rows0, rows1)
        srcb = (src0, src1)
        dstb = (dst0, dst1)
        idxb = (idx0, idx1)
        semg = (semg0, semg1)

        pltpu.sync_copy(w_hbm.at[s], nrm_v)

        for i in range(NPT // 16):
            zbuf[pl.ds(i * 16, 16)] = jnp.zeros((16,), jnp.float32)
        pltpu.sync_copy(zbuf, deg_sh.at[pl.ds(s * NPT, NPT)])
        plsc.subcore_barrier()

        def deg_body(j, _):
            pltpu.sync_copy(dst_hbm.at[s, j], dst0)
            pltpu.sync_copy(nrm_v.at[j], deg_sh.at[dst0], add=True)
            return 0
        lax.fori_loop(0, CH, deg_body, 0)
        plsc.subcore_barrier()

        pltpu.sync_copy(deg_sh.at[pl.ds(s * NPT, NPT)], zbuf)

        def dinv_body(i, _):
            d = zbuf[pl.ds(i * 16, 16)] + 1.0
            bi = lax.bitcast_convert_type(d, jnp.int32)
            y0 = lax.bitcast_convert_type(
                0x5F3759DF - lax.shift_right_arithmetic(bi, 1), jnp.float32)
            hd = 0.5 * d
            y0 = y0 * (1.5 - hd * y0 * y0)
            y0 = y0 * (1.5 - hd * y0 * y0)
            y0 = y0 * (1.5 - hd * y0 * y0)
            zbuf[pl.ds(i * 16, 16)] = y0
            dinv2_v[pl.ds(i * 16, 16)] = y0 * y0
            return 0
        lax.fori_loop(0, NPT // 16, dinv_body, 0)
        pltpu.sync_copy(zbuf, dinv_sh.at[pl.ds(s * NPT, NPT)])
        plsc.subcore_barrier()

        def nrm_body(j, _):
            pltpu.sync_copy(src_hbm.at[s, j], src0)
            pltpu.sync_copy(dst_hbm.at[s, j], dst0)
            pltpu.sync_copy(dinv_sh.at[src0], a_v)
            pltpu.sync_copy(dinv_sh.at[dst0], b_v)
            for kk in range(G // 16):
                sl = pl.ds(kk * 16, 16)
                nrm_v[j, sl] = a_v[sl] * nrm_v[j, sl] * b_v[sl]
            return 0
        lax.fori_loop(0, CH, nrm_body, 0)

        def period_body(tt, _):
            t = c * PT + tt
            tbase = t * NP
            base0 = s * NPT
            plsc.subcore_barrier()

            def init_chunk(c2, _):
                nb = base0 + c2 * G
                pltpu.sync_copy(x_hbm.at[pl.ds(tbase + nb, G)], rows0)

                def init_rows16(q, _):
                    d2 = dinv2_v[pl.ds(c2 * G + q * 16, 16)]
                    for r in range(16):
                        cf = jnp.broadcast_to(d2[r], (16,))
                        e2 = q * 16 + r
                        for kk in range(F // 16):
                            sl = pl.ds(kk * 16, 16)
                            rows0[e2, sl] = rows0[e2, sl] * cf
                    return 0
                lax.fori_loop(0, G // 16, init_rows16, 0)
                pltpu.sync_copy(rows0, acc_sh.at[pl.ds(nb, G)])
                return 0
            lax.fori_loop(0, NPT // G, init_chunk, 0)
            plsc.subcore_barrier()

            pltpu.sync_copy(src_hbm.at[s, 0], src0)
            pltpu.sync_copy(dst_hbm.at[s, 0], dst0)
            for kk in range(G // 16):
                sl = pl.ds(kk * 16, 16)
                idx0[sl] = src0[sl] + tbase
            pltpu.async_copy(x_hbm.at[idx0], rows0, semg0)

            def edge_pair(pp, _):
                for b in range(2):
                    nxt = 1 - b
                    j = pp * 2 + b

                    pltpu.make_async_copy(
                        x_hbm.at[idxb[b]], rows[b], semg[b]).wait()

                    @pl.when(j + 1 < CH)
                    def _():
                        pltpu.sync_copy(src_hbm.at[s, j + 1], srcb[nxt])
                        pltpu.sync_copy(dst_hbm.at[s, j + 1], dstb[nxt])
                        for kk in range(G // 16):
                            sl = pl.ds(kk * 16, 16)
                            idxb[nxt][sl] = srcb[nxt][sl] + tbase
                        pltpu.async_copy(
                            x_hbm.at[idxb[nxt]], rows[nxt], semg[nxt])

                    def scale_rows16(q, _):
                        nv = nrm_v[j, pl.ds(q * 16, 16)]
                        for r in range(16):
                            cf = jnp.broadcast_to(nv[r], (16,))
                            e2 = q * 16 + r
                            for kk in range(F // 16):
                                sl = pl.ds(kk * 16, 16)
                                rows[b][e2, sl] = rows[b][e2, sl] * cf
                        return 0
                    lax.fori_loop(0, G // 16, scale_rows16, 0)

                    pltpu.sync_copy(rows[b], acc_sh.at[dstb[b]], add=True)
                return 0
            lax.fori_loop(0, CH // 2, edge_pair, 0)
            plsc.subcore_barrier()

            pltpu.sync_copy(acc_sh.at[pl.ds(base0, NPT)],
                            y_hbm.at[pl.ds(tbase + base0, NPT)])
            return 0
        lax.fori_loop(0, PT, period_body, 0)

    return k(x_flat, srcr, dstr, wr)


def _tc_gates(y, attention, Wz, Wh, Lzt, Lht, bz, bh, Lzb, Lhb, Wlin, blin,
              *, interpret=False):
    NB = 1024
    grid = (NP // NB, P)

    def body(att_ref, wz_ref, wh_ref, lzt_ref, lht_ref, bz_ref, bh_ref,
             lzb_ref, lhb_ref, wlin_ref, blin_ref, y_ref, out_ref,
             az_s, ah_s, cz_s, ch_s, h_s):
        nb = pl.program_id(0)
        t = pl.program_id(1)

        @pl.when((nb == 0) & (t == 0))
        def _():
            az_s[...] = jnp.dot(wz_ref[...], lzt_ref[...],
                                preferred_element_type=jnp.float32)
            ah_s[...] = jnp.dot(wh_ref[...], lht_ref[...],
                                preferred_element_type=jnp.float32)
            cz_s[...] = jnp.dot(bz_ref[...], lzt_ref[...],
                                preferred_element_type=jnp.float32) + lzb_ref[...]
            ch_s[...] = jnp.dot(bh_ref[...], lht_ref[...],
                                preferred_element_type=jnp.float32) + lhb_ref[...]

        att = att_ref[...]
        ex = jnp.exp(att - jnp.max(att))
        pr = ex / jnp.sum(ex)
        pt = jnp.sum(jnp.where(
            lax.broadcasted_iota(jnp.int32, (1, P), 1) == t, pr, 0.0))

        yb = y_ref[0]
        z = jax.nn.sigmoid(jnp.dot(yb, az_s[...],
                                   preferred_element_type=jnp.float32)
                           + cz_s[...])
        ht = jnp.tanh(jnp.dot(yb, ah_s[...],
                              preferred_element_type=jnp.float32) + ch_s[...])
        contrib = (pt * (1.0 - z)) * ht

        @pl.when(t == 0)
        def _():
            h_s[...] = contrib

        @pl.when(t > 0)
        def _():
            h_s[...] = h_s[...] + contrib

        @pl.when(t == P - 1)
        def _():
            out_ref[...] = jnp.dot(jnp.maximum(h_s[...], 0.0), wlin_ref[...],
                                   preferred_element_type=jnp.float32) + blin_ref[...]

    return pl.pallas_call(
        body,
        grid=grid,
        in_specs=[
            pl.BlockSpec((1, P), lambda nb, t: (0, 0)),
            pl.BlockSpec((F, FOUT), lambda nb, t: (0, 0)),
            pl.BlockSpec((F, FOUT), lambda nb, t: (0, 0)),
            pl.BlockSpec((FOUT, FOUT), lambda nb, t: (0, 0)),
            pl.BlockSpec((FOUT, FOUT), lambda nb, t: (0, 0)),
            pl.BlockSpec((1, FOUT), lambda nb, t: (0, 0)),
            pl.BlockSpec((1, FOUT), lambda nb, t: (0, 0)),
            pl.BlockSpec((1, FOUT), lambda nb, t: (0, 0)),
            pl.BlockSpec((1, FOUT), lambda nb, t: (0, 0)),
            pl.BlockSpec((FOUT, P), lambda nb, t: (0, 0)),
            pl.BlockSpec((1, P), lambda nb, t: (0, 0)),
            pl.BlockSpec((1, NB, F), lambda nb, t: (t, nb, 0)),
        ],
        out_specs=pl.BlockSpec((NB, P), lambda nb, t: (nb, 0)),
        out_shape=jax.ShapeDtypeStruct((NP, P), jnp.float32),
        scratch_shapes=[
            pltpu.VMEM((F, FOUT), jnp.float32),
            pltpu.VMEM((F, FOUT), jnp.float32),
            pltpu.VMEM((1, FOUT), jnp.float32),
            pltpu.VMEM((1, FOUT), jnp.float32),
            pltpu.VMEM((NB, FOUT), jnp.float32),
        ],
        interpret=interpret,
    )(attention.reshape(1, P), Wz, Wh, Lzt, Lht,
      bz.reshape(1, FOUT), bh.reshape(1, FOUT),
      Lzb.reshape(1, FOUT), Lhb.reshape(1, FOUT),
      Wlin, blin.reshape(1, P), y)


def kernel(x, edge_index, edge_weight, attention, Wz, bz, Wr, br, Wh, bh,
           Lz_w, Lz_b, Lr_w, Lr_b, Lh_w, Lh_b, W_lin, b_lin):
    xT = jnp.transpose(x, (2, 0, 1))
    xT = jnp.pad(xT, ((0, 0), (0, NP - N), (0, 0)))
    x_flat = xT.reshape(P * NP, F)

    src = jnp.pad(edge_index[0], (0, EP - E)).reshape(NS, CH, G)
    dst = jnp.pad(edge_index[1], (0, EP - E)).reshape(NS, CH, G)
    w = jnp.pad(edge_weight, (0, EP - E)).reshape(NS, CH, G)

    y = _sc_propagate(x_flat, src, dst, w).reshape(P, NP, F)
    out = _tc_gates(y, attention, Wz, Wh, Lz_w[:FOUT], Lh_w[:FOUT],
                    bz, bh, Lz_b, Lh_b, W_lin, b_lin)
    return out[:N]


# speedup vs baseline: 24.5329x; 1.1259x over previous
import functools

import jax
import jax.numpy as jnp
from jax import lax
from jax.experimental import pallas as pl
from jax.experimental.pallas import tpu as pltpu
from jax.experimental.pallas import tpu_sc as plsc

N = 10000
NP = 10240
F = 128
P = 12
FOUT = 256
E = 160000
EP = 163840
NS = 16
NC = 2
CH = EP // NS // 128
G = 128
NPT = NP // NS
PT = P // NC


def _sc_propagate(x_flat, idxr, dstr, wr, *, interpret=False):
    mesh = plsc.VectorSubcoreMesh(core_axis_name="c", subcore_axis_name="s",
                                  num_cores=NC, num_subcores=NS)

    @functools.partial(
        pl.kernel,
        out_type=jax.ShapeDtypeStruct((P * NP, F), jnp.float32),
        mesh=mesh,
        interpret=interpret,
        compiler_params=pltpu.CompilerParams(needs_layout_passes=False),
        scratch_types=[
            pltpu.VMEM((CH, G), jnp.float32),
            pltpu.VMEM((NPT,), jnp.float32),
            pltpu.VMEM((G, F), jnp.float32),
            pltpu.VMEM((G, F), jnp.float32),
            pltpu.VMEM((G,), jnp.int32),
            pltpu.VMEM((G,), jnp.int32),
            pltpu.VMEM((G,), jnp.int32),
            pltpu.VMEM((G,), jnp.int32),
            pltpu.VMEM((G,), jnp.float32),
            pltpu.VMEM((G,), jnp.float32),
            pltpu.VMEM((NPT,), jnp.float32),
            pltpu.VMEM_SHARED((NP,), jnp.float32),
            pltpu.VMEM_SHARED((NP,), jnp.float32),
            pltpu.VMEM_SHARED((NP, F), jnp.float32),
            pltpu.SemaphoreType.DMA,
            pltpu.SemaphoreType.DMA,
            pltpu.SemaphoreType.DMA,
            pltpu.SemaphoreType.DMA,
        ],
    )
    def k(x_hbm, idx_hbm, dst_hbm, w_hbm, y_hbm,
          nrm_v, dinv2_v, rows0, rows1, dst0, dst1, idx0, idx1,
          a_v, b_v, zbuf, deg_sh, dinv_sh, acc_sh,
          semg0, semg1, semi0, semi1):
        c = lax.axis_index("c")
        s = lax.axis_index("s")
        rows = (rows0, rows1)
        dstb = (dst0, dst1)
        idxb = (idx0, idx1)
        semg = (semg0, semg1)
        semi = (semi0, semi1)

        pltpu.sync_copy(w_hbm.at[s], nrm_v)

        for i in range(NPT // 16):
            zbuf[pl.ds(i * 16, 16)] = jnp.zeros((16,), jnp.float32)
        pltpu.sync_copy(zbuf, deg_sh.at[pl.ds(s * NPT, NPT)])
        plsc.subcore_barrier()

        def deg_body(j, _):
            pltpu.sync_copy(dst_hbm.at[s, j], dst0)
            pltpu.sync_copy(nrm_v.at[j], deg_sh.at[dst0], add=True)
            return 0
        lax.fori_loop(0, CH, deg_body, 0)
        plsc.subcore_barrier()

        pltpu.sync_copy(deg_sh.at[pl.ds(s * NPT, NPT)], zbuf)

        def dinv_body(i, _):
            d = zbuf[pl.ds(i * 16, 16)] + 1.0
            bi = lax.bitcast_convert_type(d, jnp.int32)
            y0 = lax.bitcast_convert_type(
                0x5F3759DF - lax.shift_right_arithmetic(bi, 1), jnp.float32)
            hd = 0.5 * d
            y0 = y0 * (1.5 - hd * y0 * y0)
            y0 = y0 * (1.5 - hd * y0 * y0)
            y0 = y0 * (1.5 - hd * y0 * y0)
            zbuf[pl.ds(i * 16, 16)] = y0
            dinv2_v[pl.ds(i * 16, 16)] = y0 * y0
            return 0
        lax.fori_loop(0, NPT // 16, dinv_body, 0)
        pltpu.sync_copy(zbuf, dinv_sh.at[pl.ds(s * NPT, NPT)])
        plsc.subcore_barrier()

        def nrm_body(j, _):
            pltpu.sync_copy(idx_hbm.at[0, s, j], idx0)
            pltpu.sync_copy(dst_hbm.at[s, j], dst0)
            pltpu.sync_copy(dinv_sh.at[idx0], a_v)
            pltpu.sync_copy(dinv_sh.at[dst0], b_v)
            for kk in range(G // 16):
                sl = pl.ds(kk * 16, 16)
                nrm_v[j, sl] = a_v[sl] * nrm_v[j, sl] * b_v[sl]
            return 0
        lax.fori_loop(0, CH, nrm_body, 0)

        def period_body(tt, _):
            t = c * PT + tt
            tbase = t * NP
            base0 = s * NPT
            plsc.subcore_barrier()

            def init_chunk(c2, _):
                nb = base0 + c2 * G
                pltpu.sync_copy(x_hbm.at[pl.ds(tbase + nb, G)], rows0)

                def init_rows16(q, _):
                    d2 = dinv2_v[pl.ds(c2 * G + q * 16, 16)]
                    for r in range(16):
                        cf = jnp.broadcast_to(d2[r], (16,))
                        e2 = q * 16 + r
                        for kk in range(F // 16):
                            sl = pl.ds(kk * 16, 16)
                            rows0[e2, sl] = rows0[e2, sl] * cf
                    return 0
                lax.fori_loop(0, G // 16, init_rows16, 0)
                pltpu.sync_copy(rows0, acc_sh.at[pl.ds(nb, G)])
                return 0
            lax.fori_loop(0, NPT // G, init_chunk, 0)
            plsc.subcore_barrier()

            def scale_rows16_of(j, b):
                def scale_rows16(q, _):
                    nv = nrm_v[j, pl.ds(q * 16, 16)]
                    for r in range(16):
                        cf = jnp.broadcast_to(nv[r], (16,))
                        e2 = q * 16 + r
                        for kk in range(F // 16):
                            sl = pl.ds(kk * 16, 16)
                            rows[b][e2, sl] = rows[b][e2, sl] * cf
                    return 0
                lax.fori_loop(0, G // 16, scale_rows16, 0)

            def edge_step(j, b, has_next, has_next2):
                nxt = 1 - b
                pltpu.make_async_copy(
                    x_hbm.at[idxb[b]], rows[b], semg[b]).wait()
                if has_next:
                    pltpu.make_async_copy(
                        idx_hbm.at[t, s, j + 1], idxb[nxt], semi[nxt]).wait()
                    pltpu.make_async_copy(
                        dst_hbm.at[s, j + 1], dstb[nxt], semi[nxt]).wait()
                    pltpu.async_copy(
                        x_hbm.at[idxb[nxt]], rows[nxt], semg[nxt])
                scale_rows16_of(j, b)
                pltpu.sync_copy(rows[b], acc_sh.at[dstb[b]], add=True)
                if has_next2:
                    pltpu.async_copy(idx_hbm.at[t, s, j + 2], idxb[b], semi[b])
                    pltpu.async_copy(dst_hbm.at[s, j + 2], dstb[b], semi[b])

            pltpu.sync_copy(idx_hbm.at[t, s, 0], idx0)
            pltpu.sync_copy(dst_hbm.at[s, 0], dst0)
            pltpu.async_copy(x_hbm.at[idx0], rows0, semg0)
            pltpu.async_copy(idx_hbm.at[t, s, 1], idx1, semi1)
            pltpu.async_copy(dst_hbm.at[s, 1], dst1, semi1)

            def edge_pair(pp, _):
                j0 = pp * 2
                edge_step(j0, 0, True, True)
                edge_step(j0 + 1, 1, True, True)
                return 0
            lax.fori_loop(0, (CH - 2) // 2, edge_pair, 0)
            edge_step(CH - 2, 0, True, False)
            edge_step(CH - 1, 1, False, False)
            plsc.subcore_barrier()

            pltpu.sync_copy(acc_sh.at[pl.ds(base0, NPT)],
                            y_hbm.at[pl.ds(tbase + base0, NPT)])
            return 0
        lax.fori_loop(0, PT, period_body, 0)

    return k(x_flat, idxr, dstr, wr)


def _tc_gates(y, attention, Wz, Wh, Lzt, Lht, bz, bh, Lzb, Lhb, Wlin, blin,
              *, interpret=False):
    NB = 1024
    grid = (NP // NB, P)

    def body(att_ref, wz_ref, wh_ref, lzt_ref, lht_ref, bz_ref, bh_ref,
             lzb_ref, lhb_ref, wlin_ref, blin_ref, y_ref, out_ref,
             az_s, ah_s, cz_s, ch_s, h_s):
        nb = pl.program_id(0)
        t = pl.program_id(1)

        @pl.when((nb == 0) & (t == 0))
        def _():
            az_s[...] = jnp.dot(wz_ref[...], lzt_ref[...],
                                preferred_element_type=jnp.float32)
            ah_s[...] = jnp.dot(wh_ref[...], lht_ref[...],
                                preferred_element_type=jnp.float32)
            cz_s[...] = jnp.dot(bz_ref[...], lzt_ref[...],
                                preferred_element_type=jnp.float32) + lzb_ref[...]
            ch_s[...] = jnp.dot(bh_ref[...], lht_ref[...],
                                preferred_element_type=jnp.float32) + lhb_ref[...]

        att = att_ref[...]
        ex = jnp.exp(att - jnp.max(att))
        pr = ex / jnp.sum(ex)
        pt = jnp.sum(jnp.where(
            lax.broadcasted_iota(jnp.int32, (1, P), 1) == t, pr, 0.0))

        yb = y_ref[0]
        z = jax.nn.sigmoid(jnp.dot(yb, az_s[...],
                                   preferred_element_type=jnp.float32)
                           + cz_s[...])
        ht = jnp.tanh(jnp.dot(yb, ah_s[...],
                              preferred_element_type=jnp.float32) + ch_s[...])
        contrib = (pt * (1.0 - z)) * ht

        @pl.when(t == 0)
        def _():
            h_s[...] = contrib

        @pl.when(t > 0)
        def _():
            h_s[...] = h_s[...] + contrib

        @pl.when(t == P - 1)
        def _():
            out_ref[...] = jnp.dot(jnp.maximum(h_s[...], 0.0), wlin_ref[...],
                                   preferred_element_type=jnp.float32) + blin_ref[...]

    return pl.pallas_call(
        body,
        grid=grid,
        in_specs=[
            pl.BlockSpec((1, P), lambda nb, t: (0, 0)),
            pl.BlockSpec((F, FOUT), lambda nb, t: (0, 0)),
            pl.BlockSpec((F, FOUT), lambda nb, t: (0, 0)),
            pl.BlockSpec((FOUT, FOUT), lambda nb, t: (0, 0)),
            pl.BlockSpec((FOUT, FOUT), lambda nb, t: (0, 0)),
            pl.BlockSpec((1, FOUT), lambda nb, t: (0, 0)),
            pl.BlockSpec((1, FOUT), lambda nb, t: (0, 0)),
            pl.BlockSpec((1, FOUT), lambda nb, t: (0, 0)),
            pl.BlockSpec((1, FOUT), lambda nb, t: (0, 0)),
            pl.BlockSpec((FOUT, P), lambda nb, t: (0, 0)),
            pl.BlockSpec((1, P), lambda nb, t: (0, 0)),
            pl.BlockSpec((1, NB, F), lambda nb, t: (t, nb, 0)),
        ],
        out_specs=pl.BlockSpec((NB, P), lambda nb, t: (nb, 0)),
        out_shape=jax.ShapeDtypeStruct((NP, P), jnp.float32),
        scratch_shapes=[
            pltpu.VMEM((F, FOUT), jnp.float32),
            pltpu.VMEM((F, FOUT), jnp.float32),
            pltpu.VMEM((1, FOUT), jnp.float32),
            pltpu.VMEM((1, FOUT), jnp.float32),
            pltpu.VMEM((NB, FOUT), jnp.float32),
        ],
        interpret=interpret,
    )(attention.reshape(1, P), Wz, Wh, Lzt, Lht,
      bz.reshape(1, FOUT), bh.reshape(1, FOUT),
      Lzb.reshape(1, FOUT), Lhb.reshape(1, FOUT),
      Wlin, blin.reshape(1, P), y)


def kernel(x, edge_index, edge_weight, attention, Wz, bz, Wr, br, Wh, bh,
           Lz_w, Lz_b, Lr_w, Lr_b, Lh_w, Lh_b, W_lin, b_lin):
    xT = jnp.transpose(x, (2, 0, 1))
    xT = jnp.pad(xT, ((0, 0), (0, NP - N), (0, 0)))
    x_flat = xT.reshape(P * NP, F)

    src = jnp.pad(edge_index[0], (0, EP - E))
    idx_all = (src[None, :]
               + (jnp.arange(P, dtype=jnp.int32) * NP)[:, None]
               ).astype(jnp.int32).reshape(P, NS, CH, G)
    dst = jnp.pad(edge_index[1], (0, EP - E)).reshape(NS, CH, G)
    w = jnp.pad(edge_weight, (0, EP - E)).reshape(NS, CH, G)

    y = _sc_propagate(x_flat, idx_all, dst, w).reshape(P, NP, F)
    out = _tc_gates(y, attention, Wz, Wh, Lz_w[:FOUT], Lh_w[:FOUT],
                    bz, bh, Lz_b, Lh_b, W_lin, b_lin)
    return out[:N]


# speedup vs baseline: 25.5068x; 1.0397x over previous
import functools

import jax
import jax.numpy as jnp
from jax import lax
from jax.experimental import pallas as pl
from jax.experimental.pallas import tpu as pltpu
from jax.experimental.pallas import tpu_sc as plsc

N = 10000
NP = 10240
F = 128
P = 12
FOUT = 256
E = 160000
EP = 163840
NS = 16
NC = 2
CH = EP // NS // 128
G = 128
NPT = NP // NS
PT = P // NC


def _sc_propagate(x_flat, idxr, dstr, wr, *, interpret=False):
    mesh = plsc.VectorSubcoreMesh(core_axis_name="c", subcore_axis_name="s",
                                  num_cores=NC, num_subcores=NS)

    @functools.partial(
        pl.kernel,
        out_type=jax.ShapeDtypeStruct((P * NP, F), jnp.float32),
        mesh=mesh,
        interpret=interpret,
        compiler_params=pltpu.CompilerParams(needs_layout_passes=False),
        scratch_types=[
            pltpu.VMEM((CH, G), jnp.float32),
            pltpu.VMEM((NPT,), jnp.float32),
            pltpu.VMEM((G, F), jnp.float32),
            pltpu.VMEM((G, F), jnp.float32),
            pltpu.VMEM((G,), jnp.int32),
            pltpu.VMEM((G,), jnp.int32),
            pltpu.VMEM((G,), jnp.int32),
            pltpu.VMEM((G,), jnp.int32),
            pltpu.VMEM((G,), jnp.int32),
            pltpu.VMEM((G,), jnp.int32),
            pltpu.VMEM((G,), jnp.float32),
            pltpu.VMEM((G,), jnp.float32),
            pltpu.VMEM((NPT,), jnp.float32),
            pltpu.VMEM_SHARED((NP,), jnp.float32),
            pltpu.VMEM_SHARED((NP,), jnp.float32),
            pltpu.VMEM_SHARED((NP, F), jnp.float32),
            pltpu.SemaphoreType.DMA,
            pltpu.SemaphoreType.DMA,
            pltpu.SemaphoreType.DMA,
            pltpu.SemaphoreType.DMA,
            pltpu.SemaphoreType.DMA,
            pltpu.SemaphoreType.DMA,
        ],
    )
    def k(x_hbm, idx_hbm, dst_hbm, w_hbm, y_hbm,
          nrm_v, dinv2_v, rows0, rows1, dst0, dst1, idx0, idx1,
          dsts0, dsts1, a_v, b_v, zbuf, deg_sh, dinv_sh, acc_sh,
          semg0, semg1, semi0, semi1, semS0, semS1):
        c = lax.axis_index("c")
        s = lax.axis_index("s")
        rows = (rows0, rows1)
        dstb = (dst0, dst1)
        idxb = (idx0, idx1)
        semg = (semg0, semg1)
        semi = (semi0, semi1)
        semS = (semS0, semS1)
        dsts = (dsts0, dsts1)

        pltpu.sync_copy(w_hbm.at[s], nrm_v)

        for i in range(NPT // 16):
            zbuf[pl.ds(i * 16, 16)] = jnp.zeros((16,), jnp.float32)
        pltpu.sync_copy(zbuf, deg_sh.at[pl.ds(s * NPT, NPT)])
        plsc.subcore_barrier()

        def deg_body(j, _):
            pltpu.sync_copy(dst_hbm.at[s, j], dst0)
            pltpu.sync_copy(nrm_v.at[j], deg_sh.at[dst0], add=True)
            return 0
        lax.fori_loop(0, CH, deg_body, 0)
        plsc.subcore_barrier()

        pltpu.sync_copy(deg_sh.at[pl.ds(s * NPT, NPT)], zbuf)

        def dinv_body(i, _):
            d = zbuf[pl.ds(i * 16, 16)] + 1.0
            bi = lax.bitcast_convert_type(d, jnp.int32)
            y0 = lax.bitcast_convert_type(
                0x5F3759DF - lax.shift_right_arithmetic(bi, 1), jnp.float32)
            hd = 0.5 * d
            y0 = y0 * (1.5 - hd * y0 * y0)
            y0 = y0 * (1.5 - hd * y0 * y0)
            y0 = y0 * (1.5 - hd * y0 * y0)
            zbuf[pl.ds(i * 16, 16)] = y0
            dinv2_v[pl.ds(i * 16, 16)] = y0 * y0
            return 0
        lax.fori_loop(0, NPT // 16, dinv_body, 0)
        pltpu.sync_copy(zbuf, dinv_sh.at[pl.ds(s * NPT, NPT)])
        plsc.subcore_barrier()

        def nrm_body(j, _):
            pltpu.sync_copy(idx_hbm.at[0, s, j], idx0)
            pltpu.sync_copy(dst_hbm.at[s, j], dst0)
            pltpu.sync_copy(dinv_sh.at[idx0], a_v)
            pltpu.sync_copy(dinv_sh.at[dst0], b_v)
            for kk in range(G // 16):
                sl = pl.ds(kk * 16, 16)
                nrm_v[j, sl] = a_v[sl] * nrm_v[j, sl] * b_v[sl]
            return 0
        lax.fori_loop(0, CH, nrm_body, 0)

        def period_body(tt, _):
            t = c * PT + tt
            tbase = t * NP
            base0 = s * NPT
            plsc.subcore_barrier()

            def init_chunk(c2, _):
                nb = base0 + c2 * G
                pltpu.sync_copy(x_hbm.at[pl.ds(tbase + nb, G)], rows0)

                def init_rows16(q, _):
                    d2 = dinv2_v[pl.ds(c2 * G + q * 16, 16)]
                    for r in range(16):
                        cf = jnp.broadcast_to(d2[r], (16,))
                        e2 = q * 16 + r
                        for kk in range(F // 16):
                            sl = pl.ds(kk * 16, 16)
                            rows0[e2, sl] = rows0[e2, sl] * cf
                    return 0
                lax.fori_loop(0, G // 16, init_rows16, 0)
                pltpu.sync_copy(rows0, acc_sh.at[pl.ds(nb, G)])
                return 0
            lax.fori_loop(0, NPT // G, init_chunk, 0)
            plsc.subcore_barrier()

            def scale_rows16_of(j, b):
                def scale_rows16(q, _):
                    nv = nrm_v[j, pl.ds(q * 16, 16)]
                    for r in range(16):
                        cf = jnp.broadcast_to(nv[r], (16,))
                        e2 = q * 16 + r
                        for kk in range(F // 16):
                            sl = pl.ds(kk * 16, 16)
                            rows[b][e2, sl] = rows[b][e2, sl] * cf
                    return 0
                lax.fori_loop(0, G // 16, scale_rows16, 0)

            def edge_step(j, b, has_prev, has_next, has_next2):
                nxt = 1 - b
                pltpu.make_async_copy(
                    x_hbm.at[idxb[b]], rows[b], semg[b]).wait()
                if has_next:
                    pltpu.make_async_copy(
                        idx_hbm.at[t, s, j + 1], idxb[nxt], semi[nxt]).wait()
                    pltpu.make_async_copy(
                        dst_hbm.at[s, j + 1], dstb[nxt], semi[nxt]).wait()
                    if has_prev:
                        pltpu.make_async_copy(
                            rows[nxt], acc_sh.at[dsts[nxt]], semS[nxt]).wait()
                    pltpu.async_copy(
                        x_hbm.at[idxb[nxt]], rows[nxt], semg[nxt])
                scale_rows16_of(j, b)
                for kk in range(G // 16):
                    sl = pl.ds(kk * 16, 16)
                    dsts[b][sl] = dstb[b][sl]
                pltpu.async_copy(
                    rows[b], acc_sh.at[dsts[b]], semS[b], add=True)
                if has_next2:
                    pltpu.async_copy(idx_hbm.at[t, s, j + 2], idxb[b], semi[b])
                    pltpu.async_copy(dst_hbm.at[s, j + 2], dstb[b], semi[b])

            pltpu.sync_copy(idx_hbm.at[t, s, 0], idx0)
            pltpu.sync_copy(dst_hbm.at[s, 0], dst0)
            pltpu.async_copy(x_hbm.at[idx0], rows0, semg0)
            pltpu.async_copy(idx_hbm.at[t, s, 1], idx1, semi1)
            pltpu.async_copy(dst_hbm.at[s, 1], dst1, semi1)

            edge_step(0, 0, False, True, True)

            def edge_pair(pp, _):
                j0 = pp * 2 + 1
                edge_step(j0, 1, True, True, True)
                edge_step(j0 + 1, 0, True, True, True)
                return 0
            lax.fori_loop(0, (CH - 4) // 2, edge_pair, 0)
            edge_step(CH - 3, 1, True, True, True)
            edge_step(CH - 2, 0, True, True, False)
            edge_step(CH - 1, 1, True, False, False)
            pltpu.make_async_copy(
                rows1, acc_sh.at[dsts1], semS1).wait()
            plsc.subcore_barrier()

            pltpu.sync_copy(acc_sh.at[pl.ds(base0, NPT)],
                            y_hbm.at[pl.ds(tbase + base0, NPT)])
            return 0
        lax.fori_loop(0, PT, period_body, 0)

    return k(x_flat, idxr, dstr, wr)


def _tc_gates(y, attention, Wz, Wh, Lzt, Lht, bz, bh, Lzb, Lhb, Wlin, blin,
              *, interpret=False):
    NB = 1024
    grid = (NP // NB, P)

    def body(att_ref, wz_ref, wh_ref, lzt_ref, lht_ref, bz_ref, bh_ref,
             lzb_ref, lhb_ref, wlin_ref, blin_ref, y_ref, out_ref,
             az_s, ah_s, cz_s, ch_s, h_s):
        nb = pl.program_id(0)
        t = pl.program_id(1)

        @pl.when((nb == 0) & (t == 0))
        def _():
            az_s[...] = jnp.dot(wz_ref[...], lzt_ref[...],
                                preferred_element_type=jnp.float32)
            ah_s[...] = jnp.dot(wh_ref[...], lht_ref[...],
                                preferred_element_type=jnp.float32)
            cz_s[...] = jnp.dot(bz_ref[...], lzt_ref[...],
                                preferred_element_type=jnp.float32) + lzb_ref[...]
            ch_s[...] = jnp.dot(bh_ref[...], lht_ref[...],
                                preferred_element_type=jnp.float32) + lhb_ref[...]

        att = att_ref[...]
        ex = jnp.exp(att - jnp.max(att))
        pr = ex / jnp.sum(ex)
        pt = jnp.sum(jnp.where(
            lax.broadcasted_iota(jnp.int32, (1, P), 1) == t, pr, 0.0))

        yb = y_ref[0]
        z = jax.nn.sigmoid(jnp.dot(yb, az_s[...],
                                   preferred_element_type=jnp.float32)
                           + cz_s[...])
        ht = jnp.tanh(jnp.dot(yb, ah_s[...],
                              preferred_element_type=jnp.float32) + ch_s[...])
        contrib = (pt * (1.0 - z)) * ht

        @pl.when(t == 0)
        def _():
            h_s[...] = contrib

        @pl.when(t > 0)
        def _():
            h_s[...] = h_s[...] + contrib

        @pl.when(t == P - 1)
        def _():
            out_ref[...] = jnp.dot(jnp.maximum(h_s[...], 0.0), wlin_ref[...],
                                   preferred_element_type=jnp.float32) + blin_ref[...]

    return pl.pallas_call(
        body,
        grid=grid,
        in_specs=[
            pl.BlockSpec((1, P), lambda nb, t: (0, 0)),
            pl.BlockSpec((F, FOUT), lambda nb, t: (0, 0)),
            pl.BlockSpec((F, FOUT), lambda nb, t: (0, 0)),
            pl.BlockSpec((FOUT, FOUT), lambda nb, t: (0, 0)),
            pl.BlockSpec((FOUT, FOUT), lambda nb, t: (0, 0)),
            pl.BlockSpec((1, FOUT), lambda nb, t: (0, 0)),
            pl.BlockSpec((1, FOUT), lambda nb, t: (0, 0)),
            pl.BlockSpec((1, FOUT), lambda nb, t: (0, 0)),
            pl.BlockSpec((1, FOUT), lambda nb, t: (0, 0)),
            pl.BlockSpec((FOUT, P), lambda nb, t: (0, 0)),
            pl.BlockSpec((1, P), lambda nb, t: (0, 0)),
            pl.BlockSpec((1, NB, F), lambda nb, t: (t, nb, 0)),
        ],
        out_specs=pl.BlockSpec((NB, P), lambda nb, t: (nb, 0)),
        out_shape=jax.ShapeDtypeStruct((NP, P), jnp.float32),
        scratch_shapes=[
            pltpu.VMEM((F, FOUT), jnp.float32),
            pltpu.VMEM((F, FOUT), jnp.float32),
            pltpu.VMEM((1, FOUT), jnp.float32),
            pltpu.VMEM((1, FOUT), jnp.float32),
            pltpu.VMEM((NB, FOUT), jnp.float32),
        ],
        interpret=interpret,
    )(attention.reshape(1, P), Wz, Wh, Lzt, Lht,
      bz.reshape(1, FOUT), bh.reshape(1, FOUT),
      Lzb.reshape(1, FOUT), Lhb.reshape(1, FOUT),
      Wlin, blin.reshape(1, P), y)


def kernel(x, edge_index, edge_weight, attention, Wz, bz, Wr, br, Wh, bh,
           Lz_w, Lz_b, Lr_w, Lr_b, Lh_w, Lh_b, W_lin, b_lin):
    xT = jnp.transpose(x, (2, 0, 1))
    xT = jnp.pad(xT, ((0, 0), (0, NP - N), (0, 0)))
    x_flat = xT.reshape(P * NP, F)

    src = jnp.pad(edge_index[0], (0, EP - E))
    idx_all = (src[None, :]
               + (jnp.arange(P, dtype=jnp.int32) * NP)[:, None]
               ).astype(jnp.int32).reshape(P, NS, CH, G)
    dst = jnp.pad(edge_index[1], (0, EP - E)).reshape(NS, CH, G)
    w = jnp.pad(edge_weight, (0, EP - E)).reshape(NS, CH, G)

    y = _sc_propagate(x_flat, idx_all, dst, w).reshape(P, NP, F)
    out = _tc_gates(y, attention, Wz, Wh, Lz_w[:FOUT], Lh_w[:FOUT],
                    bz, bh, Lz_b, Lh_b, W_lin, b_lin)
    return out[:N]
